# baseline (speedup 1.0000x reference)
import functools

import jax
import jax.numpy as jnp
from jax.experimental import pallas as pl
from jax.experimental.pallas import tpu as pltpu
from jax.experimental.pallas import tpu_sc as plsc

_N = 8
_M = 1024
_D = 32
_HL = 16
_B = 256
_T = _B * _HL
_CHUNK = 2048
_NC = _T // _CHUNK
_CC = 0.25
_TOT = _N * _T * _D

_NW = 32
_ROWS_PW = (_N * _T) // _NW
_IDX_CHUNK = 128
_NJ = _ROWS_PW // _IDX_CHUNK


def _tc_body(xb_ref, eb_ref, xsq_ref, esq_ref,
             gidx_ref, scal_ref, counts_ref, iota_ref, acc_ref):
    n = pl.program_id(0)
    c = pl.program_id(1)

    @pl.when(jnp.logical_and(n == 0, c == 0))
    def _init():
        acc_ref[0] = 0.0
        acc_ref[1] = 0.0
        acc_ref[2] = 0.0
        iota_ref[...] = jax.lax.broadcasted_iota(
            jnp.int32, (1, _M), 1).astype(jnp.float32)

    @pl.when(c == 0)
    def _zero_counts():
        counts_ref[...] = jnp.zeros_like(counts_ref)

    xb = xb_ref[0, 0]
    eb = eb_ref[0]
    s = jax.lax.dot_general(
        xb, eb, (((1,), (1,)), ((), ())),
        preferred_element_type=jnp.float32)
    d = (esq_ref[0] + xsq_ref[0, 0]) - 2.0 * s

    vmin = jnp.min(d, axis=1, keepdims=True)
    eq = d == vmin
    idx_f = jnp.min(jnp.where(eq, iota_ref[...], float(_M)),
                    axis=1, keepdims=True)
    gidx_ref[0, 0] = idx_f.astype(jnp.int32) + n * _M

    counts_ref[...] += jnp.sum(eq.astype(jnp.float32), axis=0, keepdims=True)
    acc_ref[0] += jnp.sum(vmin)

    @pl.when(c == _NC - 1)
    def _entropy():
        p = counts_ref[...] * (1.0 / _T)
        ent = -jnp.sum(p * jnp.log(p + 1e-10))
        acc_ref[1] += ent
        acc_ref[2] += jnp.exp(ent)

    @pl.when(jnp.logical_and(n == _N - 1, c == _NC - 1))
    def _emit():
        scal_ref[0] = _CC * acc_ref[0] / _TOT
        scal_ref[1] = acc_ref[1] / _N
        scal_ref[2] = acc_ref[2] / _N


def _vq_tc(xb, eb, xsq, esq):
    return pl.pallas_call(
        _tc_body,
        grid=(_N, _NC),
        in_specs=[
            pl.BlockSpec((1, 1, _CHUNK, _D), lambda n, c: (n, c, 0, 0)),
            pl.BlockSpec((1, _M, _D), lambda n, c: (n, 0, 0)),
            pl.BlockSpec((1, 1, _CHUNK, 1), lambda n, c: (n, c, 0, 0)),
            pl.BlockSpec((1, 1, _M), lambda n, c: (n, 0, 0)),
        ],
        out_specs=[
            pl.BlockSpec((1, 1, _CHUNK, 1), lambda n, c: (n, c, 0, 0)),
            pl.BlockSpec(memory_space=pltpu.SMEM),
        ],
        out_shape=[
            jax.ShapeDtypeStruct((_N, _NC, _CHUNK, 1), jnp.int32),
            jax.ShapeDtypeStruct((3,), jnp.float32),
        ],
        scratch_shapes=[
            pltpu.VMEM((1, _M), jnp.float32),
            pltpu.VMEM((1, _M), jnp.float32),
            pltpu.SMEM((3,), jnp.float32),
        ],
        compiler_params=pltpu.CompilerParams(
            dimension_semantics=("arbitrary", "arbitrary")),
    )(xb, eb, xsq, esq)


_SC_MESH = plsc.VectorSubcoreMesh(core_axis_name="c", subcore_axis_name="s")


_TQ = 4
_TPT = _T // _TQ
_GRP = _TPT // _HL


@functools.partial(
    pl.kernel,
    mesh=_SC_MESH,
    out_type=jax.ShapeDtypeStruct((_B, _N * _D * _HL), jnp.float32),
    scratch_types=[
        pltpu.VMEM((_M * _D,), jnp.float32),
        pltpu.VMEM((_TPT,), jnp.int32),
        pltpu.VMEM((_GRP, _D * _HL), jnp.float32),
    ],
    compiler_params=pltpu.CompilerParams(use_tc_tiling_on_sc=False,
                                         needs_layout_passes=False),
)
def _sc_zq(table_hbm, gidx_hbm, out_hbm, tab_v, idx_v, stage_v):
    wid = jax.lax.axis_index("s") * 2 + jax.lax.axis_index("c")
    n = wid // _TQ
    tq = wid - n * _TQ
    pltpu.sync_copy(table_hbm.at[pl.ds(n * _M * _D, _M * _D)], tab_v)
    pltpu.sync_copy(gidx_hbm.at[n, pl.ds(tq * _TPT, _TPT)], idx_v)

    def body(i, carry):
        rows16 = idx_v[pl.ds(i * _HL, _HL)]
        base = rows16 * _D - n * (_M * _D)
        for dd in range(_D):
            vals = plsc.load_gather(tab_v, [base + dd])
            stage_v[i, pl.ds(dd * _HL, _HL)] = vals
        return carry

    jax.lax.fori_loop(0, _GRP, body, 0)
    pltpu.sync_copy(stage_v,
                    out_hbm.at[pl.ds(tq * _GRP, _GRP),
                               pl.ds(n * _D * _HL, _D * _HL)])


def kernel(x, embedding):
    bs = x.shape[0]
    xr = x.reshape(bs, _N, _D, _HL)
    xf = xr.transpose(1, 0, 3, 2).reshape(_N, _T, _D)
    xb = xf.astype(jnp.bfloat16).reshape(_N, _NC, _CHUNK, _D)
    eb = embedding.astype(jnp.bfloat16)
    xsq = (jnp.sum(xr * xr, axis=2).transpose(1, 0, 2)
           .reshape(_N, _NC, _CHUNK, 1))
    esq = jnp.sum(embedding * embedding, axis=2).reshape(_N, 1, _M)

    gidx4, scal = _vq_tc(xb, eb, xsq, esq)

    gidx = gidx4.reshape(_N, _T)
    z_q = _sc_zq(embedding.reshape(_N * _M * _D), gidx)

    indices = gidx - (jnp.arange(_N, dtype=jnp.int32) * _M)[:, None]
    indices_out = indices.reshape(_N, _B, _HL, 1).transpose(1, 0, 2, 3)
    return (z_q, scal[0], scal[1], scal[2], indices_out)


# speedup vs baseline: 1.0173x; 1.0173x over previous
import functools

import jax
import jax.numpy as jnp
from jax.experimental import pallas as pl
from jax.experimental.pallas import tpu as pltpu
from jax.experimental.pallas import tpu_sc as plsc

_N = 8
_M = 1024
_D = 32
_HL = 16
_B = 256
_T = _B * _HL
_CHUNK = 2048
_NC = _T // _CHUNK
_CC = 0.25
_TOT = _N * _T * _D

_NW = 32
_ROWS_PW = (_N * _T) // _NW
_IDX_CHUNK = 128
_NJ = _ROWS_PW // _IDX_CHUNK


def _tc_body(xb_ref, eb_ref, xsq_ref, esq_ref,
             gidx_ref, scal_ref, counts_ref, iota_ref, acc_ref):
    n = pl.program_id(0)
    c = pl.program_id(1)

    @pl.when(jnp.logical_and(n == 0, c == 0))
    def _init():
        acc_ref[0] = 0.0
        acc_ref[1] = 0.0
        acc_ref[2] = 0.0
        iota_ref[...] = jax.lax.broadcasted_iota(
            jnp.int32, (1, _M), 1).astype(jnp.float32)

    @pl.when(c == 0)
    def _zero_counts():
        counts_ref[...] = jnp.zeros_like(counts_ref)

    xb = xb_ref[0, 0]
    eb = eb_ref[0]
    s = jax.lax.dot_general(
        xb, eb, (((1,), (1,)), ((), ())),
        preferred_element_type=jnp.float32)
    d = (esq_ref[0] + xsq_ref[0, 0]) - 2.0 * s

    vmin = jnp.min(d, axis=1, keepdims=True)
    eq = d == vmin
    idx_f = jnp.min(jnp.where(eq, iota_ref[...], float(_M)),
                    axis=1, keepdims=True)
    gidx_ref[0, 0] = idx_f.astype(jnp.int32) + n * _M

    counts_ref[...] += jnp.sum(eq.astype(jnp.float32), axis=0, keepdims=True)
    acc_ref[0] += jnp.sum(vmin)

    @pl.when(c == _NC - 1)
    def _entropy():
        p = counts_ref[...] * (1.0 / _T)
        ent = -jnp.sum(p * jnp.log(p + 1e-10))
        acc_ref[1] += ent
        acc_ref[2] += jnp.exp(ent)

    @pl.when(jnp.logical_and(n == _N - 1, c == _NC - 1))
    def _emit():
        scal_ref[0] = _CC * acc_ref[0] / _TOT
        scal_ref[1] = acc_ref[1] / _N
        scal_ref[2] = acc_ref[2] / _N


def _vq_tc(xb, eb, xsq, esq):
    return pl.pallas_call(
        _tc_body,
        grid=(_N, _NC),
        in_specs=[
            pl.BlockSpec((1, 1, _CHUNK, _D), lambda n, c: (n, c, 0, 0)),
            pl.BlockSpec((1, _M, _D), lambda n, c: (n, 0, 0)),
            pl.BlockSpec((1, 1, _CHUNK, 1), lambda n, c: (n, c, 0, 0)),
            pl.BlockSpec((1, 1, _M), lambda n, c: (n, 0, 0)),
        ],
        out_specs=[
            pl.BlockSpec((1, 1, _CHUNK, 1), lambda n, c: (n, c, 0, 0)),
            pl.BlockSpec(memory_space=pltpu.SMEM),
        ],
        out_shape=[
            jax.ShapeDtypeStruct((_N, _NC, _CHUNK, 1), jnp.int32),
            jax.ShapeDtypeStruct((3,), jnp.float32),
        ],
        scratch_shapes=[
            pltpu.VMEM((1, _M), jnp.float32),
            pltpu.VMEM((1, _M), jnp.float32),
            pltpu.SMEM((3,), jnp.float32),
        ],
        compiler_params=pltpu.CompilerParams(
            dimension_semantics=("arbitrary", "arbitrary"),
            allow_input_fusion=[True, True, True, True]),
    )(xb, eb, xsq, esq)


_SC_MESH = plsc.VectorSubcoreMesh(core_axis_name="c", subcore_axis_name="s")


@functools.partial(
    pl.kernel,
    mesh=_SC_MESH,
    out_type=jax.ShapeDtypeStruct((_N * _T, _D), jnp.float32),
    scratch_types=[
        pltpu.VMEM((_NJ, _IDX_CHUNK), jnp.int32),
        pltpu.VMEM((_ROWS_PW, _D), jnp.float32),
        pltpu.SemaphoreType.DMA,
    ],
    compiler_params=pltpu.CompilerParams(use_tc_tiling_on_sc=False),
)
def _sc_gather(table_hbm, gidx_hbm, out_hbm, idx_v, rows_v, sem):
    wid = jax.lax.axis_index("s") * 2 + jax.lax.axis_index("c")
    pltpu.sync_copy(gidx_hbm.at[wid], idx_v)
    copies = [
        pltpu.async_copy(table_hbm.at[idx_v.at[j]],
                         rows_v.at[pl.ds(j * _IDX_CHUNK, _IDX_CHUNK)], sem)
        for j in range(_NJ)
    ]
    for cp in copies:
        cp.wait()
    pltpu.sync_copy(rows_v, out_hbm.at[pl.ds(wid * _ROWS_PW, _ROWS_PW)])


def kernel(x, embedding):
    bs = x.shape[0]
    xr = x.reshape(bs, _N, _D, _HL)
    xf = xr.transpose(1, 0, 3, 2).reshape(_N, _T, _D)
    xb = xf.astype(jnp.bfloat16).reshape(_N, _NC, _CHUNK, _D)
    eb = embedding.astype(jnp.bfloat16)
    xsq = (jnp.sum(xr * xr, axis=2).transpose(1, 0, 2)
           .reshape(_N, _NC, _CHUNK, 1))
    esq = jnp.sum(embedding * embedding, axis=2).reshape(_N, 1, _M)

    gidx4, scal = _vq_tc(xb, eb, xsq, esq)

    gidx = gidx4.reshape(_N, _T)
    q = _sc_gather(embedding.reshape(_N * _M, _D),
                   gidx.reshape(_NW, _NJ, _IDX_CHUNK))

    indices = gidx - (jnp.arange(_N, dtype=jnp.int32) * _M)[:, None]
    z_q = (q.reshape(_N, _B, _HL, _D).transpose(1, 0, 3, 2)
           .reshape(bs, _N * _D * _HL))
    indices_out = indices.reshape(_N, _B, _HL, 1).transpose(1, 0, 2, 3)
    return (z_q, scal[0], scal[1], scal[2], indices_out)


# speedup vs baseline: 1.0390x; 1.0213x over previous
import functools

import jax
import jax.numpy as jnp
from jax.experimental import pallas as pl
from jax.experimental.pallas import tpu as pltpu
from jax.experimental.pallas import tpu_sc as plsc

_N = 8
_M = 1024
_D = 32
_HL = 16
_B = 256
_T = _B * _HL
_CHUNK = 4096
_NC = _T // _CHUNK
_CC = 0.25
_TOT = _N * _T * _D

_NW = 32
_ROWS_PW = (_N * _T) // _NW
_IDX_CHUNK = 128
_NJ = _ROWS_PW // _IDX_CHUNK


def _tc_body(xb_ref, eb_ref, xsq_ref, esq_ref,
             gidx_ref, scal_ref, counts_ref, iota_ref, acc_ref):
    n = pl.program_id(0)
    c = pl.program_id(1)

    @pl.when(jnp.logical_and(n == 0, c == 0))
    def _init():
        acc_ref[0] = 0.0
        acc_ref[1] = 0.0
        acc_ref[2] = 0.0
        iota_ref[...] = jax.lax.broadcasted_iota(
            jnp.int32, (1, _M), 1).astype(jnp.float32)

    @pl.when(c == 0)
    def _zero_counts():
        counts_ref[...] = jnp.zeros_like(counts_ref)

    xb = xb_ref[0, 0]
    eb = eb_ref[0]
    s = jax.lax.dot_general(
        xb, eb, (((1,), (1,)), ((), ())),
        preferred_element_type=jnp.float32)
    d = (esq_ref[0] + xsq_ref[0, 0]) - 2.0 * s

    vmin = jnp.min(d, axis=1, keepdims=True)
    eq = d == vmin
    idx_f = jnp.min(jnp.where(eq, iota_ref[...], float(_M)),
                    axis=1, keepdims=True)
    gidx_ref[0, 0] = idx_f.astype(jnp.int32) + n * _M

    counts_ref[...] += jnp.sum(eq.astype(jnp.float32), axis=0, keepdims=True)
    acc_ref[0] += jnp.sum(vmin)

    @pl.when(c == _NC - 1)
    def _entropy():
        p = counts_ref[...] * (1.0 / _T)
        ent = -jnp.sum(p * jnp.log(p + 1e-10))
        acc_ref[1] += ent
        acc_ref[2] += jnp.exp(ent)

    @pl.when(jnp.logical_and(n == _N - 1, c == _NC - 1))
    def _emit():
        scal_ref[0] = _CC * acc_ref[0] / _TOT
        scal_ref[1] = acc_ref[1] / _N
        scal_ref[2] = acc_ref[2] / _N


def _vq_tc(xb, eb, xsq, esq):
    return pl.pallas_call(
        _tc_body,
        grid=(_N, _NC),
        in_specs=[
            pl.BlockSpec((1, 1, _CHUNK, _D), lambda n, c: (n, c, 0, 0)),
            pl.BlockSpec((1, _M, _D), lambda n, c: (n, 0, 0)),
            pl.BlockSpec((1, 1, _CHUNK, 1), lambda n, c: (n, c, 0, 0)),
            pl.BlockSpec((1, 1, _M), lambda n, c: (n, 0, 0)),
        ],
        out_specs=[
            pl.BlockSpec((1, 1, _CHUNK, 1), lambda n, c: (n, c, 0, 0)),
            pl.BlockSpec(memory_space=pltpu.SMEM),
        ],
        out_shape=[
            jax.ShapeDtypeStruct((_N, _NC, _CHUNK, 1), jnp.int32),
            jax.ShapeDtypeStruct((3,), jnp.float32),
        ],
        scratch_shapes=[
            pltpu.VMEM((1, _M), jnp.float32),
            pltpu.VMEM((1, _M), jnp.float32),
            pltpu.SMEM((3,), jnp.float32),
        ],
        compiler_params=pltpu.CompilerParams(
            dimension_semantics=("arbitrary", "arbitrary"),
            allow_input_fusion=[True, True, True, True]),
    )(xb, eb, xsq, esq)


_SC_MESH = plsc.VectorSubcoreMesh(core_axis_name="c", subcore_axis_name="s")


@functools.partial(
    pl.kernel,
    mesh=_SC_MESH,
    out_type=jax.ShapeDtypeStruct((_N * _T, _D), jnp.float32),
    scratch_types=[
        pltpu.VMEM((_NJ, _IDX_CHUNK), jnp.int32),
        pltpu.VMEM((_ROWS_PW, _D), jnp.float32),
        pltpu.SemaphoreType.DMA,
    ],
    compiler_params=pltpu.CompilerParams(use_tc_tiling_on_sc=False),
)
def _sc_gather(table_hbm, gidx_hbm, out_hbm, idx_v, rows_v, sem):
    wid = jax.lax.axis_index("s") * 2 + jax.lax.axis_index("c")
    pltpu.sync_copy(gidx_hbm.at[wid], idx_v)
    copies = [
        pltpu.async_copy(table_hbm.at[idx_v.at[j]],
                         rows_v.at[pl.ds(j * _IDX_CHUNK, _IDX_CHUNK)], sem)
        for j in range(_NJ)
    ]
    for cp in copies:
        cp.wait()
    pltpu.sync_copy(rows_v, out_hbm.at[pl.ds(wid * _ROWS_PW, _ROWS_PW)])


def kernel(x, embedding):
    bs = x.shape[0]
    xr = x.reshape(bs, _N, _D, _HL)
    xf = xr.transpose(1, 0, 3, 2).reshape(_N, _T, _D)
    xb = xf.astype(jnp.bfloat16).reshape(_N, _NC, _CHUNK, _D)
    eb = embedding.astype(jnp.bfloat16)
    xsq = (jnp.sum(xr * xr, axis=2).transpose(1, 0, 2)
           .reshape(_N, _NC, _CHUNK, 1))
    esq = jnp.sum(embedding * embedding, axis=2).reshape(_N, 1, _M)

    gidx4, scal = _vq_tc(xb, eb, xsq, esq)

    gidx = gidx4.reshape(_N, _T)
    q = _sc_gather(embedding.reshape(_N * _M, _D),
                   gidx.reshape(_NW, _NJ, _IDX_CHUNK))

    indices = gidx - (jnp.arange(_N, dtype=jnp.int32) * _M)[:, None]
    z_q = (q.reshape(_N, _B, _HL, _D).transpose(1, 0, 3, 2)
           .reshape(bs, _N * _D * _HL))
    indices_out = indices.reshape(_N, _B, _HL, 1).transpose(1, 0, 2, 3)
    return (z_q, scal[0], scal[1], scal[2], indices_out)
